# baseline (speedup 1.0000x reference)
import functools
import jax
import jax.numpy as jnp
from jax import lax
from jax.experimental import pallas as pl
from jax.experimental.pallas import tpu as pltpu
from jax.experimental.pallas import tpu_sc as plsc

_CHUNK = 32


def kernel(x, weight):
    B, S, H = x.shape
    w = weight[:S]
    NW = 32
    total_rows = B * S
    rpw = total_rows // NW
    chunk_words = _CHUNK * H
    x1 = x.reshape(total_rows * H)
    w1 = w.reshape(S * H)
    mesh = plsc.VectorSubcoreMesh(core_axis_name="c", subcore_axis_name="s")

    @functools.partial(
        pl.kernel,
        mesh=mesh,
        out_type=jax.ShapeDtypeStruct((total_rows * H,), jnp.float32),
        scratch_types=[
            pltpu.VMEM((chunk_words,), jnp.float32),
            pltpu.VMEM((chunk_words,), jnp.float32),
        ],
    )
    def run(x_hbm, w_hbm, o_hbm, xbuf, wbuf):
        c = lax.axis_index("c")
        s_idx = lax.axis_index("s")
        wid = s_idx * 2 + c
        row0 = wid * rpw
        xoff = row0 * H
        woff = (row0 % S) * H

        def step(t, carry):
            xo = xoff + t * chunk_words
            wo = woff + t * chunk_words
            pltpu.sync_copy(x_hbm.at[pl.ds(xo, chunk_words)], xbuf)
            pltpu.sync_copy(w_hbm.at[pl.ds(wo, chunk_words)], wbuf)

            def inner(i, carry2):
                sl = pl.ds(i * 16, 16)
                xbuf[sl] = xbuf[sl] + wbuf[sl]
                return carry2

            lax.fori_loop(0, chunk_words // 16, inner, 0)
            pltpu.sync_copy(xbuf, o_hbm.at[pl.ds(xo, chunk_words)])
            return carry

        lax.fori_loop(0, rpw // _CHUNK, step, 0)

    out = run(x1, w1)
    return out.reshape(B, S, H)


# speedup vs baseline: 1.7805x; 1.7805x over previous
import functools
import jax
import jax.numpy as jnp
from jax import lax
from jax.experimental import pallas as pl
from jax.experimental.pallas import tpu as pltpu
from jax.experimental.pallas import tpu_sc as plsc

_CHUNK = 16
_UNROLL = 8


def kernel(x, weight):
    B, S, H = x.shape
    w = weight[:S]
    NW = 32
    total_rows = B * S
    rpw = total_rows // NW
    cw = _CHUNK * H
    nch = rpw // _CHUNK
    x1 = x.reshape(total_rows * H)
    w1 = w.reshape(S * H)
    mesh = plsc.VectorSubcoreMesh(core_axis_name="c", subcore_axis_name="s")

    vmem = pltpu.VMEM((cw,), jnp.float32)

    @functools.partial(
        pl.kernel,
        mesh=mesh,
        out_type=jax.ShapeDtypeStruct((total_rows * H,), jnp.float32),
        scratch_types=[
            vmem, vmem, vmem, vmem, vmem, vmem,
            pltpu.SemaphoreType.DMA,
            pltpu.SemaphoreType.DMA,
            pltpu.SemaphoreType.DMA,
            pltpu.SemaphoreType.DMA,
        ],
    )
    def run(x_hbm, w_hbm, o_hbm, xb0, xb1, wb0, wb1, ob0, ob1,
            si0, si1, so0, so1):
        c = lax.axis_index("c")
        s_idx = lax.axis_index("s")
        wid = s_idx * 2 + c
        row0 = wid * rpw
        xoff = row0 * H
        woff = (row0 % S) * H
        xbufs = (xb0, xb1)
        wbufs = (wb0, wb1)
        obufs = (ob0, ob1)
        sin = (si0, si1)
        sout = (so0, so1)

        def start_in(t, b):
            pltpu.async_copy(x_hbm.at[pl.ds(xoff + t * cw, cw)], xbufs[b], sin[b])
            pltpu.async_copy(w_hbm.at[pl.ds(woff + t * cw, cw)], wbufs[b], sin[b])

        def wait_in(t, b):
            pltpu.make_async_copy(
                x_hbm.at[pl.ds(xoff + t * cw, cw)], xbufs[b], sin[b]).wait()
            pltpu.make_async_copy(
                w_hbm.at[pl.ds(woff + t * cw, cw)], wbufs[b], sin[b]).wait()

        def start_out(t, b):
            pltpu.async_copy(obufs[b], o_hbm.at[pl.ds(xoff + t * cw, cw)], sout[b])

        def wait_out(t, b):
            pltpu.make_async_copy(
                obufs[b], o_hbm.at[pl.ds(xoff + t * cw, cw)], sout[b]).wait()

        def compute(b):
            xb, wb, ob = xbufs[b], wbufs[b], obufs[b]

            def inner(i, carry):
                base = i * (16 * _UNROLL)
                for u in range(_UNROLL):
                    sl = pl.ds(base + u * 16, 16)
                    ob[sl] = xb[sl] + wb[sl]
                return carry

            lax.fori_loop(0, cw // (16 * _UNROLL), inner, 0)

        start_in(0, 0)
        start_in(1, 1)

        def step(t2, carry):
            for b in range(2):
                t = t2 * 2 + b
                wait_in(t, b)
                compute(b)
                @pl.when(t2 > 0)
                def _():
                    wait_out(t - 2, b)
                start_out(t, b)
                @pl.when(t2 < nch // 2 - 1)
                def _():
                    start_in(t + 2, b)
            return carry

        lax.fori_loop(0, nch // 2, step, 0)
        wait_out(nch - 2, 0)
        wait_out(nch - 1, 1)

    out = run(x1, w1)
    return out.reshape(B, S, H)
